# Initial kernel scaffold; baseline (speedup 1.0000x reference)
import functools

import jax
import jax.numpy as jnp
from jax import lax
from jax.experimental import pallas as pl
from jax.experimental.pallas import tpu as pltpu
from jax.experimental.pallas import tpu_sc as plsc

_N = 10000
_E = 320000
_K = 128
_NC = 2
_NS = 16
_L = 16
_NB = 500


def _make_edge_phase(F, H):
    R = F + _L
    CHUNKS = _E // _K
    W = _NC * _NS
    CPW = -(-CHUNKS // W)
    ROWS_T = _N // _NS
    ZR = 125
    FH = F // H
    mesh = plsc.VectorSubcoreMesh(core_axis_name="c", subcore_axis_name="s")

    @functools.partial(
        pl.kernel,
        out_type=jax.ShapeDtypeStruct((_NC, _N, R), jnp.float32),
        mesh=mesh,
        scratch_types=[
            pltpu.VMEM((2 * H, _N), jnp.float32),
            pltpu.VMEM((_K,), jnp.int32),
            pltpu.VMEM((_K,), jnp.int32),
            pltpu.VMEM((_K, F), jnp.float32),
            pltpu.VMEM((_K, R), jnp.float32),
            pltpu.VMEM((H, _K), jnp.float32),
            pltpu.VMEM((125, F + _L), jnp.float32),
            pltpu.VMEM_SHARED((_N, F + _L), jnp.float32),
            pltpu.SemaphoreType.DMA,
        ],
    )
    def edge_phase(h_hbm, edges_hbm, table_hbm, out_hbm,
                   table_v, src_v, dst_v, rows_v, msg_v, p_v, zero_v,
                   accum_sh, sem):
        core = lax.axis_index("c")
        sub = lax.axis_index("s")
        wid = sub * _NC + core

        pltpu.sync_copy(table_hbm, table_v)

        @pl.loop(0, ZR)
        def _(r):
            @pl.loop(0, R, step=_L)
            def _(cidx):
                zero_v[r, pl.ds(cidx, _L)] = jnp.zeros((_L,), jnp.float32)

        @pl.loop(0, ROWS_T // ZR)
        def _(i):
            pltpu.sync_copy(zero_v,
                            accum_sh.at[pl.ds(sub * ROWS_T + i * ZR, ZR)])
        plsc.subcore_barrier()

        lane = lax.iota(jnp.int32, _L)

        @pl.loop(0, CPW)
        def _(i):
            c = wid + i * W

            @pl.when(c < CHUNKS)
            def _():
                off = c * _K
                pltpu.sync_copy(edges_hbm.at[0, pl.ds(off, _K)], src_v)
                pltpu.sync_copy(edges_hbm.at[1, pl.ds(off, _K)], dst_v)
                pltpu.async_copy(h_hbm.at[src_v], rows_v, sem).wait()

                for g in range(_K // _L):
                    sidx = src_v[pl.ds(g * _L, _L)]
                    didx = dst_v[pl.ds(g * _L, _L)]
                    for h in range(H):
                        asv = plsc.load_gather(
                            table_v, [jnp.full((_L,), h, jnp.int32), sidx])
                        adv = plsc.load_gather(
                            table_v, [jnp.full((_L,), H + h, jnp.int32), didx])
                        e = asv + adv
                        e = jnp.maximum(e, 0.2 * e)
                        p_v[h, pl.ds(g * _L, _L)] = jnp.exp(e)

                @pl.loop(0, _K)
                def _(j):
                    tail = jnp.zeros((_L,), jnp.float32)
                    for h in range(H):
                        phv = jnp.full((_L,), p_v[h, j])
                        for v in range(FH // _L):
                            col = h * FH + v * _L
                            msg_v[j, pl.ds(col, _L)] = (
                                rows_v[j, pl.ds(col, _L)] * phv)
                        tail = tail + jnp.where(lane == h, phv, 0.0)
                    msg_v[j, pl.ds(F, _L)] = tail

                pltpu.sync_copy(msg_v, accum_sh.at[dst_v], add=True)

        plsc.subcore_barrier()
        pltpu.sync_copy(accum_sh.at[pl.ds(sub * ROWS_T, ROWS_T)],
                        out_hbm.at[core, pl.ds(sub * ROWS_T, ROWS_T)])

    return edge_phase


_edge1 = _make_edge_phase(128, 2)
_edge2 = _make_edge_phase(64, 1)


def _tc1(x, W1, a_src1, a_dst1):
    def body(x_ref, w_ref, as_ref, ad_ref, h_ref, sc_ref):
        h = jnp.dot(x_ref[...], w_ref[...],
                    preferred_element_type=jnp.float32)
        h_ref[...] = h
        cols = []
        for hh in range(2):
            cols.append(jnp.sum(h[:, hh * 64:(hh + 1) * 64]
                                * as_ref[hh][None, :], axis=1))
        for hh in range(2):
            cols.append(jnp.sum(h[:, hh * 64:(hh + 1) * 64]
                                * ad_ref[hh][None, :], axis=1))
        sc_ref[...] = jnp.stack(cols, axis=0)

    return pl.pallas_call(
        body,
        grid=(_N // _NB,),
        in_specs=[pl.BlockSpec((_NB, 128), lambda i: (i, 0)),
                  pl.BlockSpec((128, 128), lambda i: (0, 0)),
                  pl.BlockSpec((2, 64), lambda i: (0, 0)),
                  pl.BlockSpec((2, 64), lambda i: (0, 0))],
        out_specs=[pl.BlockSpec((_NB, 128), lambda i: (i, 0)),
                   pl.BlockSpec((4, _NB), lambda i: (0, i))],
        out_shape=[jax.ShapeDtypeStruct((_N, 128), jnp.float32),
                   jax.ShapeDtypeStruct((4, _N), jnp.float32)],
    )(x, W1, a_src1, a_dst1)


def _tc2(acc, b1, W2, a_src2, a_dst2):
    def body(a_ref, b1_ref, w_ref, as_ref, ad_ref, h_ref, sc_ref):
        g = a_ref[0] + a_ref[1]
        parts = []
        for hh in range(2):
            num = g[:, hh * 64:(hh + 1) * 64]
            den = g[:, 128 + hh:129 + hh]
            parts.append(num / (den + 1e-16))
        out1 = jnp.concatenate(parts, axis=1) + b1_ref[...]
        r = jnp.maximum(out1, 0.0)
        h2 = jnp.dot(r, w_ref[...], preferred_element_type=jnp.float32)
        h_ref[...] = h2
        sc_ref[...] = jnp.stack(
            [jnp.sum(h2 * as_ref[0][None, :], axis=1),
             jnp.sum(h2 * ad_ref[0][None, :], axis=1)], axis=0)

    return pl.pallas_call(
        body,
        grid=(_N // _NB,),
        in_specs=[pl.BlockSpec((2, _NB, 144), lambda i: (0, i, 0)),
                  pl.BlockSpec((1, 128), lambda i: (0, 0)),
                  pl.BlockSpec((128, 64), lambda i: (0, 0)),
                  pl.BlockSpec((1, 64), lambda i: (0, 0)),
                  pl.BlockSpec((1, 64), lambda i: (0, 0))],
        out_specs=[pl.BlockSpec((_NB, 64), lambda i: (i, 0)),
                   pl.BlockSpec((2, _NB), lambda i: (0, i))],
        out_shape=[jax.ShapeDtypeStruct((_N, 64), jnp.float32),
                   jax.ShapeDtypeStruct((2, _N), jnp.float32)],
    )(acc, b1, W2, a_src2, a_dst2)


def _tc3(acc, b2):
    def body(a_ref, b2_ref, z_ref):
        g = a_ref[0] + a_ref[1]
        z_ref[...] = g[:, 0:64] / (g[:, 64:65] + 1e-16) + b2_ref[...]

    return pl.pallas_call(
        body,
        grid=(_N // _NB,),
        in_specs=[pl.BlockSpec((2, _NB, 80), lambda i: (0, i, 0)),
                  pl.BlockSpec((1, 64), lambda i: (0, 0))],
        out_specs=pl.BlockSpec((_NB, 64), lambda i: (i, 0)),
        out_shape=jax.ShapeDtypeStruct((_N, 64), jnp.float32),
    )(acc, b2)


def kernel(x, edge_index, W1, a_src1, a_dst1, b1, W2, a_src2, a_dst2, b2):
    h1, sc1 = _tc1(x, W1, a_src1, a_dst1)
    acc1 = _edge1(h1, edge_index, sc1)
    h2, sc2 = _tc2(acc1, b1.reshape(1, 128), W2, a_src2, a_dst2)
    acc2 = _edge2(h2, edge_index, sc2)
    return _tc3(acc2, b2.reshape(1, 64))


# speedup vs baseline: 29.2606x; 29.2606x over previous
import functools

import jax
import jax.numpy as jnp
from jax import lax
from jax.experimental import pallas as pl
from jax.experimental.pallas import tpu as pltpu
from jax.experimental.pallas import tpu_sc as plsc

_N = 10000
_E = 320000
_K = 128
_NC = 2
_NS = 16
_L = 16
_NB = 1000


def _vgather(v, idx):
    dnums = lax.GatherDimensionNumbers(
        offset_dims=(), collapsed_slice_dims=(0,), start_index_map=(0,))
    return lax.gather(v, idx[:, None], dnums, (1,),
                      mode=lax.GatherScatterMode.PROMISE_IN_BOUNDS)


def _make_edge_phase(F, H):
    CHUNKS = _E // _K
    W = _NC * _NS
    CPW = -(-CHUNKS // W)
    ROWS_T = _N // _NS
    ZR = 125
    FH = F // H
    mesh = plsc.VectorSubcoreMesh(core_axis_name="c", subcore_axis_name="s",
                                  num_cores=_NC, num_subcores=_NS)

    @functools.partial(
        pl.kernel,
        out_type=(jax.ShapeDtypeStruct((_NC, _N, F), jnp.float32),
                  jax.ShapeDtypeStruct((_NC, _N, _L), jnp.float32)),
        mesh=mesh,
        scratch_types=[
            pltpu.VMEM((_K,), jnp.int32),
            pltpu.VMEM((_K,), jnp.int32),
            pltpu.VMEM((_K, F), jnp.float32),
            pltpu.VMEM((_K, _L), jnp.float32),
            pltpu.VMEM((_K, _L), jnp.float32),
            pltpu.VMEM((_K, _L), jnp.float32),
            pltpu.VMEM_SHARED((_N, F), jnp.float32),
            pltpu.VMEM_SHARED((_N, _L), jnp.float32),
            pltpu.SemaphoreType.DMA,
        ],
        compiler_params=pltpu.CompilerParams(use_tc_tiling_on_sc=False),
    )
    def edge_phase(h_hbm, edges_hbm, table_hbm, out_hbm, s_hbm,
                   src_v, dst_v, rows_v, srow_v, drow_v, sp_v,
                   accum_sh, s_sh, sem):
        core = lax.axis_index("c")
        sub = lax.axis_index("s")
        wid = sub * _NC + core
        lane = lax.iota(jnp.int32, _L)

        @pl.loop(0, _K)
        def _(r):
            @pl.loop(0, F, step=_L)
            def _(cidx):
                rows_v[r, pl.ds(cidx, _L)] = jnp.zeros((_L,), jnp.float32)
            sp_v[r, pl.ds(0, _L)] = jnp.zeros((_L,), jnp.float32)

        @pl.loop(0, ROWS_T // ZR)
        def _(i):
            pltpu.sync_copy(rows_v.at[pl.ds(0, ZR)],
                            accum_sh.at[pl.ds(sub * ROWS_T + i * ZR, ZR)])
            pltpu.sync_copy(sp_v.at[pl.ds(0, ZR)],
                            s_sh.at[pl.ds(sub * ROWS_T + i * ZR, ZR)])
        plsc.subcore_barrier()

        shift_idx = jnp.minimum(lane + H, _L - 1)
        pmask = lane < H

        @pl.loop(0, CPW)
        def _(i):
            c = wid + i * W

            @pl.when(c < CHUNKS)
            def _():
                off = c * _K
                pltpu.sync_copy(edges_hbm.at[0, pl.ds(off, _K)], src_v)
                pltpu.sync_copy(edges_hbm.at[1, pl.ds(off, _K)], dst_v)
                pltpu.async_copy(h_hbm.at[src_v], rows_v, sem).wait()
                pltpu.async_copy(table_hbm.at[src_v], srow_v, sem).wait()
                pltpu.async_copy(table_hbm.at[dst_v], drow_v, sem).wait()

                @pl.loop(0, _K)
                def _(j):
                    sv = srow_v[j, pl.ds(0, _L)]
                    dv = drow_v[j, pl.ds(0, _L)]
                    dsh = _vgather(dv, shift_idx)
                    e = sv + dsh
                    e = jnp.maximum(e, 0.2 * e)
                    pe = jnp.exp(e)
                    for h in range(H):
                        phv = _vgather(pe, jnp.full((_L,), h, jnp.int32))
                        for v in range(FH // _L):
                            col = h * FH + v * _L
                            rows_v[j, pl.ds(col, _L)] = (
                                rows_v[j, pl.ds(col, _L)] * phv)
                    sp_v[j, pl.ds(0, _L)] = jnp.where(pmask, pe, 0.0)

                pltpu.sync_copy(rows_v, accum_sh.at[dst_v], add=True)
                pltpu.sync_copy(sp_v, s_sh.at[dst_v], add=True)

        plsc.subcore_barrier()
        pltpu.sync_copy(accum_sh.at[pl.ds(sub * ROWS_T, ROWS_T)],
                        out_hbm.at[core, pl.ds(sub * ROWS_T, ROWS_T)])
        pltpu.sync_copy(s_sh.at[pl.ds(sub * ROWS_T, ROWS_T)],
                        s_hbm.at[core, pl.ds(sub * ROWS_T, ROWS_T)])

    return edge_phase


_edge1 = _make_edge_phase(128, 2)
_edge2 = _make_edge_phase(64, 1)


def _tc1(x, W1, a_src1, a_dst1):
    def body(x_ref, w_ref, as_ref, ad_ref, h_ref, sc_ref):
        h = jnp.dot(x_ref[...], w_ref[...],
                    preferred_element_type=jnp.float32)
        h_ref[...] = h
        cols = []
        for hh in range(2):
            cols.append(jnp.sum(h[:, hh * 64:(hh + 1) * 64]
                                * as_ref[hh][None, :], axis=1))
        for hh in range(2):
            cols.append(jnp.sum(h[:, hh * 64:(hh + 1) * 64]
                                * ad_ref[hh][None, :], axis=1))
        tab = jnp.stack(cols, axis=1)
        sc_ref[...] = jnp.concatenate(
            [tab, jnp.zeros((tab.shape[0], 12), jnp.float32)], axis=1)

    return pl.pallas_call(
        body,
        grid=(_N // _NB,),
        in_specs=[pl.BlockSpec((_NB, 128), lambda i: (i, 0)),
                  pl.BlockSpec((128, 128), lambda i: (0, 0)),
                  pl.BlockSpec((2, 64), lambda i: (0, 0)),
                  pl.BlockSpec((2, 64), lambda i: (0, 0))],
        out_specs=[pl.BlockSpec((_NB, 128), lambda i: (i, 0)),
                   pl.BlockSpec((_NB, 16), lambda i: (i, 0))],
        out_shape=[jax.ShapeDtypeStruct((_N, 128), jnp.float32),
                   jax.ShapeDtypeStruct((_N, 16), jnp.float32)],
    )(x, W1, a_src1, a_dst1)


def _tc2(acc, sden, b1, W2, a_src2, a_dst2):
    def body(a_ref, s_ref, b1_ref, w_ref, as_ref, ad_ref, h_ref, sc_ref):
        g = a_ref[0] + a_ref[1]
        sd = s_ref[0] + s_ref[1]
        parts = []
        for hh in range(2):
            num = g[:, hh * 64:(hh + 1) * 64]
            den = sd[:, hh:hh + 1]
            parts.append(num / (den + 1e-16))
        out1 = jnp.concatenate(parts, axis=1) + b1_ref[...]
        r = jnp.maximum(out1, 0.0)
        h2 = jnp.dot(r, w_ref[...], preferred_element_type=jnp.float32)
        h_ref[...] = h2
        tab = jnp.stack(
            [jnp.sum(h2 * as_ref[0][None, :], axis=1),
             jnp.sum(h2 * ad_ref[0][None, :], axis=1)], axis=1)
        sc_ref[...] = jnp.concatenate(
            [tab, jnp.zeros((tab.shape[0], 14), jnp.float32)], axis=1)

    return pl.pallas_call(
        body,
        grid=(_N // _NB,),
        in_specs=[pl.BlockSpec((2, _NB, 128), lambda i: (0, i, 0)),
                  pl.BlockSpec((2, _NB, 16), lambda i: (0, i, 0)),
                  pl.BlockSpec((1, 128), lambda i: (0, 0)),
                  pl.BlockSpec((128, 64), lambda i: (0, 0)),
                  pl.BlockSpec((1, 64), lambda i: (0, 0)),
                  pl.BlockSpec((1, 64), lambda i: (0, 0))],
        out_specs=[pl.BlockSpec((_NB, 64), lambda i: (i, 0)),
                   pl.BlockSpec((_NB, 16), lambda i: (i, 0))],
        out_shape=[jax.ShapeDtypeStruct((_N, 64), jnp.float32),
                   jax.ShapeDtypeStruct((_N, 16), jnp.float32)],
    )(acc, sden, b1, W2, a_src2, a_dst2)


def _tc3(acc, sden, b2):
    def body(a_ref, s_ref, b2_ref, z_ref):
        g = a_ref[0] + a_ref[1]
        sd = s_ref[0] + s_ref[1]
        z_ref[...] = g / (sd[:, 0:1] + 1e-16) + b2_ref[...]

    return pl.pallas_call(
        body,
        grid=(_N // _NB,),
        in_specs=[pl.BlockSpec((2, _NB, 64), lambda i: (0, i, 0)),
                  pl.BlockSpec((2, _NB, 16), lambda i: (0, i, 0)),
                  pl.BlockSpec((1, 64), lambda i: (0, 0))],
        out_specs=pl.BlockSpec((_NB, 64), lambda i: (i, 0)),
        out_shape=jax.ShapeDtypeStruct((_N, 64), jnp.float32),
    )(acc, sden, b2)


def kernel(x, edge_index, W1, a_src1, a_dst1, b1, W2, a_src2, a_dst2, b2):
    h1, tab1 = _tc1(x, W1, a_src1, a_dst1)
    acc1, s1 = _edge1(h1, edge_index, tab1)
    h2, tab2 = _tc2(acc1, s1, b1.reshape(1, 128), W2, a_src2, a_dst2)
    acc2, s2 = _edge2(h2, edge_index, tab2)
    return _tc3(acc2, s2, b2.reshape(1, 64))


# speedup vs baseline: 40.5631x; 1.3863x over previous
import functools

import jax
import jax.numpy as jnp
from jax import lax
from jax.experimental import pallas as pl
from jax.experimental.pallas import tpu as pltpu
from jax.experimental.pallas import tpu_sc as plsc

_N = 10000
_E = 320000
_K = 80
_NC = 2
_NS = 16
_L = 16
_NB = 1000


def _vgather(v, idx):
    dnums = lax.GatherDimensionNumbers(
        offset_dims=(), collapsed_slice_dims=(0,), start_index_map=(0,))
    return lax.gather(v, idx[:, None], dnums, (1,),
                      mode=lax.GatherScatterMode.PROMISE_IN_BOUNDS)


def _make_edge_phase(F, H):
    CHUNKS = _E // _K
    W = _NC * _NS
    CPW = -(-CHUNKS // W)
    CPH = -(-CPW // 2)
    ROWS_T = _N // _NS
    ZR = 25
    FH = F // H
    mesh = plsc.VectorSubcoreMesh(core_axis_name="c", subcore_axis_name="s",
                                  num_cores=_NC, num_subcores=_NS)

    @functools.partial(
        pl.kernel,
        out_type=(jax.ShapeDtypeStruct((_NC, _N, F), jnp.float32),
                  jax.ShapeDtypeStruct((_NC, _N, _L), jnp.float32)),
        mesh=mesh,
        scratch_types=[
            pltpu.VMEM((_K,), jnp.int32),
            pltpu.VMEM((_K,), jnp.int32),
            pltpu.VMEM((_K,), jnp.int32),
            pltpu.VMEM((_K,), jnp.int32),
            pltpu.VMEM((_K, F), jnp.float32),
            pltpu.VMEM((_K, F), jnp.float32),
            pltpu.VMEM((_K, _L), jnp.float32),
            pltpu.VMEM((_K, _L), jnp.float32),
            pltpu.VMEM((_K, _L), jnp.float32),
            pltpu.VMEM((_K, _L), jnp.float32),
            pltpu.VMEM_SHARED((_N, F), jnp.float32),
            pltpu.VMEM_SHARED((_N, _L), jnp.float32),
            pltpu.SemaphoreType.DMA,
            pltpu.SemaphoreType.DMA,
        ],
        compiler_params=pltpu.CompilerParams(use_tc_tiling_on_sc=False),
    )
    def edge_phase(h_hbm, edges_hbm, table_hbm, out_hbm, s_hbm,
                   src0_v, dst0_v, src1_v, dst1_v, rows0_v, rows1_v,
                   srow0_v, srow1_v, drow0_v, drow1_v,
                   accum_sh, s_sh, sem0, sem1):
        core = lax.axis_index("c")
        sub = lax.axis_index("s")
        wid = sub * _NC + core
        lane = lax.iota(jnp.int32, _L)
        bufs = ((src0_v, dst0_v, rows0_v, srow0_v, drow0_v, sem0),
                (src1_v, dst1_v, rows1_v, srow1_v, drow1_v, sem1))

        @pl.loop(0, _K)
        def _(r):
            @pl.loop(0, F, step=_L)
            def _(cidx):
                rows0_v[r, pl.ds(cidx, _L)] = jnp.zeros((_L,), jnp.float32)
            srow0_v[r, pl.ds(0, _L)] = jnp.zeros((_L,), jnp.float32)

        @pl.loop(0, ROWS_T // ZR)
        def _(i):
            pltpu.sync_copy(rows0_v.at[pl.ds(0, ZR)],
                            accum_sh.at[pl.ds(sub * ROWS_T + i * ZR, ZR)])
            pltpu.sync_copy(srow0_v.at[pl.ds(0, ZR)],
                            s_sh.at[pl.ds(sub * ROWS_T + i * ZR, ZR)])
        plsc.subcore_barrier()

        shift_idx = jnp.minimum(lane + H, _L - 1)
        pmask = lane < H

        def start(c, s):
            src_v, dst_v, rows_v, srow_v, drow_v, sem = bufs[s]
            off = c * _K
            pltpu.sync_copy(edges_hbm.at[0, pl.ds(off, _K)], src_v)
            pltpu.sync_copy(edges_hbm.at[1, pl.ds(off, _K)], dst_v)
            pltpu.async_copy(h_hbm.at[src_v], rows_v, sem)
            pltpu.async_copy(table_hbm.at[src_v], srow_v, sem)
            pltpu.async_copy(table_hbm.at[dst_v], drow_v, sem)

        def finish(c, s):
            src_v, dst_v, rows_v, srow_v, drow_v, sem = bufs[s]
            pltpu.make_async_copy(h_hbm.at[src_v], rows_v, sem).wait()
            pltpu.make_async_copy(table_hbm.at[src_v], srow_v, sem).wait()
            pltpu.make_async_copy(table_hbm.at[dst_v], drow_v, sem).wait()

            @pl.loop(0, _K)
            def _(j):
                sv = srow_v[j, pl.ds(0, _L)]
                dv = drow_v[j, pl.ds(0, _L)]
                dsh = _vgather(dv, shift_idx)
                e = sv + dsh
                e = jnp.maximum(e, 0.2 * e)
                pe = jnp.exp(e)
                for h in range(H):
                    phv = _vgather(pe, jnp.full((_L,), h, jnp.int32))
                    for v in range(FH // _L):
                        col = h * FH + v * _L
                        rows_v[j, pl.ds(col, _L)] = (
                            rows_v[j, pl.ds(col, _L)] * phv)
                srow_v[j, pl.ds(0, _L)] = jnp.where(pmask, pe, 0.0)

            pltpu.sync_copy(rows_v, accum_sh.at[dst_v], add=True)
            pltpu.sync_copy(srow_v, s_sh.at[dst_v], add=True)

        @pl.when(wid < CHUNKS)
        def _():
            start(wid, 0)

        @pl.loop(0, CPH)
        def _(t):
            c0 = wid + (2 * t) * W
            c1 = c0 + W
            c2 = c1 + W

            @pl.when(c1 < CHUNKS)
            def _():
                start(c1, 1)

            @pl.when(c0 < CHUNKS)
            def _():
                finish(c0, 0)

            @pl.when(c2 < CHUNKS)
            def _():
                start(c2, 0)

            @pl.when(c1 < CHUNKS)
            def _():
                finish(c1, 1)

        plsc.subcore_barrier()
        pltpu.sync_copy(accum_sh.at[pl.ds(sub * ROWS_T, ROWS_T)],
                        out_hbm.at[core, pl.ds(sub * ROWS_T, ROWS_T)])
        pltpu.sync_copy(s_sh.at[pl.ds(sub * ROWS_T, ROWS_T)],
                        s_hbm.at[core, pl.ds(sub * ROWS_T, ROWS_T)])

    return edge_phase


_edge1 = _make_edge_phase(128, 2)
_edge2 = _make_edge_phase(64, 1)


def _tc1(x, W1, a_src1, a_dst1):
    def body(x_ref, w_ref, as_ref, ad_ref, h_ref, sc_ref):
        h = jnp.dot(x_ref[...], w_ref[...],
                    preferred_element_type=jnp.float32)
        h_ref[...] = h
        cols = []
        for hh in range(2):
            cols.append(jnp.sum(h[:, hh * 64:(hh + 1) * 64]
                                * as_ref[hh][None, :], axis=1))
        for hh in range(2):
            cols.append(jnp.sum(h[:, hh * 64:(hh + 1) * 64]
                                * ad_ref[hh][None, :], axis=1))
        tab = jnp.stack(cols, axis=1)
        sc_ref[...] = jnp.concatenate(
            [tab, jnp.zeros((tab.shape[0], 12), jnp.float32)], axis=1)

    return pl.pallas_call(
        body,
        grid=(_N // _NB,),
        in_specs=[pl.BlockSpec((_NB, 128), lambda i: (i, 0)),
                  pl.BlockSpec((128, 128), lambda i: (0, 0)),
                  pl.BlockSpec((2, 64), lambda i: (0, 0)),
                  pl.BlockSpec((2, 64), lambda i: (0, 0))],
        out_specs=[pl.BlockSpec((_NB, 128), lambda i: (i, 0)),
                   pl.BlockSpec((_NB, 16), lambda i: (i, 0))],
        out_shape=[jax.ShapeDtypeStruct((_N, 128), jnp.float32),
                   jax.ShapeDtypeStruct((_N, 16), jnp.float32)],
    )(x, W1, a_src1, a_dst1)


def _tc2(acc, sden, b1, W2, a_src2, a_dst2):
    def body(a_ref, s_ref, b1_ref, w_ref, as_ref, ad_ref, h_ref, sc_ref):
        g = a_ref[0] + a_ref[1]
        sd = s_ref[0] + s_ref[1]
        parts = []
        for hh in range(2):
            num = g[:, hh * 64:(hh + 1) * 64]
            den = sd[:, hh:hh + 1]
            parts.append(num / (den + 1e-16))
        out1 = jnp.concatenate(parts, axis=1) + b1_ref[...]
        r = jnp.maximum(out1, 0.0)
        h2 = jnp.dot(r, w_ref[...], preferred_element_type=jnp.float32)
        h_ref[...] = h2
        tab = jnp.stack(
            [jnp.sum(h2 * as_ref[0][None, :], axis=1),
             jnp.sum(h2 * ad_ref[0][None, :], axis=1)], axis=1)
        sc_ref[...] = jnp.concatenate(
            [tab, jnp.zeros((tab.shape[0], 14), jnp.float32)], axis=1)

    return pl.pallas_call(
        body,
        grid=(_N // _NB,),
        in_specs=[pl.BlockSpec((2, _NB, 128), lambda i: (0, i, 0)),
                  pl.BlockSpec((2, _NB, 16), lambda i: (0, i, 0)),
                  pl.BlockSpec((1, 128), lambda i: (0, 0)),
                  pl.BlockSpec((128, 64), lambda i: (0, 0)),
                  pl.BlockSpec((1, 64), lambda i: (0, 0)),
                  pl.BlockSpec((1, 64), lambda i: (0, 0))],
        out_specs=[pl.BlockSpec((_NB, 64), lambda i: (i, 0)),
                   pl.BlockSpec((_NB, 16), lambda i: (i, 0))],
        out_shape=[jax.ShapeDtypeStruct((_N, 64), jnp.float32),
                   jax.ShapeDtypeStruct((_N, 16), jnp.float32)],
    )(acc, sden, b1, W2, a_src2, a_dst2)


def _tc3(acc, sden, b2):
    def body(a_ref, s_ref, b2_ref, z_ref):
        g = a_ref[0] + a_ref[1]
        sd = s_ref[0] + s_ref[1]
        z_ref[...] = g / (sd[:, 0:1] + 1e-16) + b2_ref[...]

    return pl.pallas_call(
        body,
        grid=(_N // _NB,),
        in_specs=[pl.BlockSpec((2, _NB, 64), lambda i: (0, i, 0)),
                  pl.BlockSpec((2, _NB, 16), lambda i: (0, i, 0)),
                  pl.BlockSpec((1, 64), lambda i: (0, 0))],
        out_specs=pl.BlockSpec((_NB, 64), lambda i: (i, 0)),
        out_shape=jax.ShapeDtypeStruct((_N, 64), jnp.float32),
    )(acc, sden, b2)


def kernel(x, edge_index, W1, a_src1, a_dst1, b1, W2, a_src2, a_dst2, b2):
    h1, tab1 = _tc1(x, W1, a_src1, a_dst1)
    acc1, s1 = _edge1(h1, edge_index, tab1)
    h2, tab2 = _tc2(acc1, s1, b1.reshape(1, 128), W2, a_src2, a_dst2)
    acc2, s2 = _edge2(h2, edge_index, tab2)
    return _tc3(acc2, s2, b2.reshape(1, 64))


# speedup vs baseline: 46.0050x; 1.1342x over previous
import functools

import jax
import jax.numpy as jnp
from jax import lax
from jax.experimental import pallas as pl
from jax.experimental.pallas import tpu as pltpu
from jax.experimental.pallas import tpu_sc as plsc

_N = 10000
_E = 320000
_K = 80
_NC = 2
_NS = 16
_L = 16
_NB = 1000


def _vgather(v, idx):
    dnums = lax.GatherDimensionNumbers(
        offset_dims=(), collapsed_slice_dims=(0,), start_index_map=(0,))
    return lax.gather(v, idx[:, None], dnums, (1,),
                      mode=lax.GatherScatterMode.PROMISE_IN_BOUNDS)


def _make_edge_phase(F, H):
    CHUNKS = _E // _K
    W = _NC * _NS
    CPW = -(-CHUNKS // W)
    CPH = -(-CPW // 2)
    ROWS_T = _N // _NS
    ZR = 25
    FH = F // H
    mesh = plsc.VectorSubcoreMesh(core_axis_name="c", subcore_axis_name="s",
                                  num_cores=_NC, num_subcores=_NS)

    @functools.partial(
        pl.kernel,
        out_type=(jax.ShapeDtypeStruct((_NC, _N, F), jnp.float32),
                  jax.ShapeDtypeStruct((_NC, _N, _L), jnp.float32)),
        mesh=mesh,
        scratch_types=[
            pltpu.VMEM((2, _K), jnp.int32),
            pltpu.VMEM((2, _K), jnp.int32),
            pltpu.VMEM((_K, F), jnp.float32),
            pltpu.VMEM((_K, F), jnp.float32),
            pltpu.VMEM((_K, _L), jnp.float32),
            pltpu.VMEM((_K, _L), jnp.float32),
            pltpu.VMEM((_K, _L), jnp.float32),
            pltpu.VMEM((_K, _L), jnp.float32),
            pltpu.VMEM_SHARED((_N, F), jnp.float32),
            pltpu.VMEM_SHARED((_N, _L), jnp.float32),
            pltpu.SemaphoreType.DMA,
            pltpu.SemaphoreType.DMA,
        ],
        compiler_params=pltpu.CompilerParams(use_tc_tiling_on_sc=False),
    )
    def edge_phase(h_hbm, edges_hbm, table_hbm, out_hbm, s_hbm,
                   eidx0_v, eidx1_v, rows0_v, rows1_v,
                   srow0_v, srow1_v, drow0_v, drow1_v,
                   accum_sh, s_sh, sem0, sem1):
        core = lax.axis_index("c")
        sub = lax.axis_index("s")
        wid = sub * _NC + core
        lane = lax.iota(jnp.int32, _L)
        bufs = ((eidx0_v, rows0_v, srow0_v, drow0_v, sem0),
                (eidx1_v, rows1_v, srow1_v, drow1_v, sem1))

        @pl.loop(0, _K)
        def _(r):
            @pl.loop(0, F, step=_L)
            def _(cidx):
                rows0_v[r, pl.ds(cidx, _L)] = jnp.zeros((_L,), jnp.float32)
            srow0_v[r, pl.ds(0, _L)] = jnp.zeros((_L,), jnp.float32)

        @pl.loop(0, ROWS_T // ZR)
        def _(i):
            pltpu.sync_copy(rows0_v.at[pl.ds(0, ZR)],
                            accum_sh.at[pl.ds(sub * ROWS_T + i * ZR, ZR)])
            pltpu.sync_copy(srow0_v.at[pl.ds(0, ZR)],
                            s_sh.at[pl.ds(sub * ROWS_T + i * ZR, ZR)])
        plsc.subcore_barrier()

        shift_idx = jnp.minimum(lane + H, _L - 1)
        pmask = lane < H

        def start(c, s):
            eidx_v, rows_v, srow_v, drow_v, sem = bufs[s]
            off = c * _K
            pltpu.sync_copy(edges_hbm.at[:, pl.ds(off, _K)], eidx_v)
            src_v = eidx_v.at[0]
            dst_v = eidx_v.at[1]
            pltpu.async_copy(h_hbm.at[src_v], rows_v, sem)
            pltpu.async_copy(table_hbm.at[src_v], srow_v, sem)
            pltpu.async_copy(table_hbm.at[dst_v], drow_v, sem)

        def finish(c, s):
            eidx_v, rows_v, srow_v, drow_v, sem = bufs[s]
            src_v = eidx_v.at[0]
            dst_v = eidx_v.at[1]
            pltpu.make_async_copy(h_hbm.at[src_v], rows_v, sem).wait()
            pltpu.make_async_copy(table_hbm.at[src_v], srow_v, sem).wait()
            pltpu.make_async_copy(table_hbm.at[dst_v], drow_v, sem).wait()

            @pl.loop(0, _K)
            def _(j):
                sv = srow_v[j, pl.ds(0, _L)]
                dv = drow_v[j, pl.ds(0, _L)]
                dsh = _vgather(dv, shift_idx)
                e = sv + dsh
                e = jnp.maximum(e, 0.2 * e)
                pe = jnp.exp(e)
                for h in range(H):
                    phv = _vgather(pe, jnp.full((_L,), h, jnp.int32))
                    for v in range(FH // _L):
                        col = h * FH + v * _L
                        rows_v[j, pl.ds(col, _L)] = (
                            rows_v[j, pl.ds(col, _L)] * phv)
                srow_v[j, pl.ds(0, _L)] = jnp.where(pmask, pe, 0.0)

            pltpu.sync_copy(rows_v, accum_sh.at[dst_v], add=True)
            pltpu.sync_copy(srow_v, s_sh.at[dst_v], add=True)

        @pl.when(wid < CHUNKS)
        def _():
            start(wid, 0)

        @pl.loop(0, CPH)
        def _(t):
            c0 = wid + (2 * t) * W
            c1 = c0 + W
            c2 = c1 + W

            @pl.when(c1 < CHUNKS)
            def _():
                start(c1, 1)

            @pl.when(c0 < CHUNKS)
            def _():
                finish(c0, 0)

            @pl.when(c2 < CHUNKS)
            def _():
                start(c2, 0)

            @pl.when(c1 < CHUNKS)
            def _():
                finish(c1, 1)

        plsc.subcore_barrier()
        pltpu.sync_copy(accum_sh.at[pl.ds(sub * ROWS_T, ROWS_T)],
                        out_hbm.at[core, pl.ds(sub * ROWS_T, ROWS_T)])
        pltpu.sync_copy(s_sh.at[pl.ds(sub * ROWS_T, ROWS_T)],
                        s_hbm.at[core, pl.ds(sub * ROWS_T, ROWS_T)])

    return edge_phase


_edge1 = _make_edge_phase(128, 2)
_edge2 = _make_edge_phase(64, 1)


def _tc1(x, W1, a_src1, a_dst1):
    def body(x_ref, w_ref, as_ref, ad_ref, h_ref, sc_ref):
        h = jnp.dot(x_ref[...], w_ref[...],
                    preferred_element_type=jnp.float32)
        h_ref[...] = h
        cols = []
        for hh in range(2):
            cols.append(jnp.sum(h[:, hh * 64:(hh + 1) * 64]
                                * as_ref[hh][None, :], axis=1))
        for hh in range(2):
            cols.append(jnp.sum(h[:, hh * 64:(hh + 1) * 64]
                                * ad_ref[hh][None, :], axis=1))
        tab = jnp.stack(cols, axis=1)
        sc_ref[...] = jnp.concatenate(
            [tab, jnp.zeros((tab.shape[0], 12), jnp.float32)], axis=1)

    return pl.pallas_call(
        body,
        grid=(_N // _NB,),
        in_specs=[pl.BlockSpec((_NB, 128), lambda i: (i, 0)),
                  pl.BlockSpec((128, 128), lambda i: (0, 0)),
                  pl.BlockSpec((2, 64), lambda i: (0, 0)),
                  pl.BlockSpec((2, 64), lambda i: (0, 0))],
        out_specs=[pl.BlockSpec((_NB, 128), lambda i: (i, 0)),
                   pl.BlockSpec((_NB, 16), lambda i: (i, 0))],
        out_shape=[jax.ShapeDtypeStruct((_N, 128), jnp.float32),
                   jax.ShapeDtypeStruct((_N, 16), jnp.float32)],
    )(x, W1, a_src1, a_dst1)


def _tc2(acc, sden, b1, W2, a_src2, a_dst2):
    def body(a_ref, s_ref, b1_ref, w_ref, as_ref, ad_ref, h_ref, sc_ref):
        g = a_ref[0] + a_ref[1]
        sd = s_ref[0] + s_ref[1]
        parts = []
        for hh in range(2):
            num = g[:, hh * 64:(hh + 1) * 64]
            den = sd[:, hh:hh + 1]
            parts.append(num / (den + 1e-16))
        out1 = jnp.concatenate(parts, axis=1) + b1_ref[...]
        r = jnp.maximum(out1, 0.0)
        h2 = jnp.dot(r, w_ref[...], preferred_element_type=jnp.float32)
        h_ref[...] = h2
        tab = jnp.stack(
            [jnp.sum(h2 * as_ref[0][None, :], axis=1),
             jnp.sum(h2 * ad_ref[0][None, :], axis=1)], axis=1)
        sc_ref[...] = jnp.concatenate(
            [tab, jnp.zeros((tab.shape[0], 14), jnp.float32)], axis=1)

    return pl.pallas_call(
        body,
        grid=(_N // _NB,),
        in_specs=[pl.BlockSpec((2, _NB, 128), lambda i: (0, i, 0)),
                  pl.BlockSpec((2, _NB, 16), lambda i: (0, i, 0)),
                  pl.BlockSpec((1, 128), lambda i: (0, 0)),
                  pl.BlockSpec((128, 64), lambda i: (0, 0)),
                  pl.BlockSpec((1, 64), lambda i: (0, 0)),
                  pl.BlockSpec((1, 64), lambda i: (0, 0))],
        out_specs=[pl.BlockSpec((_NB, 64), lambda i: (i, 0)),
                   pl.BlockSpec((_NB, 16), lambda i: (i, 0))],
        out_shape=[jax.ShapeDtypeStruct((_N, 64), jnp.float32),
                   jax.ShapeDtypeStruct((_N, 16), jnp.float32)],
    )(acc, sden, b1, W2, a_src2, a_dst2)


def _tc3(acc, sden, b2):
    def body(a_ref, s_ref, b2_ref, z_ref):
        g = a_ref[0] + a_ref[1]
        sd = s_ref[0] + s_ref[1]
        z_ref[...] = g / (sd[:, 0:1] + 1e-16) + b2_ref[...]

    return pl.pallas_call(
        body,
        grid=(_N // _NB,),
        in_specs=[pl.BlockSpec((2, _NB, 64), lambda i: (0, i, 0)),
                  pl.BlockSpec((2, _NB, 16), lambda i: (0, i, 0)),
                  pl.BlockSpec((1, 64), lambda i: (0, 0))],
        out_specs=pl.BlockSpec((_NB, 64), lambda i: (i, 0)),
        out_shape=jax.ShapeDtypeStruct((_N, 64), jnp.float32),
    )(acc, sden, b2)


def kernel(x, edge_index, W1, a_src1, a_dst1, b1, W2, a_src2, a_dst2, b2):
    h1, tab1 = _tc1(x, W1, a_src1, a_dst1)
    acc1, s1 = _edge1(h1, edge_index, tab1)
    h2, tab2 = _tc2(acc1, s1, b1.reshape(1, 128), W2, a_src2, a_dst2)
    acc2, s2 = _edge2(h2, edge_index, tab2)
    return _tc3(acc2, s2, b2.reshape(1, 64))


# speedup vs baseline: 46.7645x; 1.0165x over previous
import functools

import jax
import jax.numpy as jnp
from jax import lax
from jax.experimental import pallas as pl
from jax.experimental.pallas import tpu as pltpu
from jax.experimental.pallas import tpu_sc as plsc

_N = 10000
_E = 320000
_K = 80
_NC = 2
_NS = 16
_L = 16
_NB = 1000


def _vgather(v, idx):
    dnums = lax.GatherDimensionNumbers(
        offset_dims=(), collapsed_slice_dims=(0,), start_index_map=(0,))
    return lax.gather(v, idx[:, None], dnums, (1,),
                      mode=lax.GatherScatterMode.PROMISE_IN_BOUNDS)


def _make_edge_phase(F, H):
    FT = F + _L
    CHUNKS = _E // _K
    W = _NC * _NS
    CPW = -(-CHUNKS // W)
    CPH = -(-CPW // 2)
    ROWS_T = _N // _NS
    ZR = 25
    FH = F // H
    mesh = plsc.VectorSubcoreMesh(core_axis_name="c", subcore_axis_name="s",
                                  num_cores=_NC, num_subcores=_NS)

    @functools.partial(
        pl.kernel,
        out_type=jax.ShapeDtypeStruct((_NC, _N, FT), jnp.float32),
        mesh=mesh,
        scratch_types=[
            pltpu.VMEM((2, _K), jnp.int32),
            pltpu.VMEM((2, _K), jnp.int32),
            pltpu.VMEM((_K, FT), jnp.float32),
            pltpu.VMEM((_K, FT), jnp.float32),
            pltpu.VMEM((_K, _L), jnp.float32),
            pltpu.VMEM((_K, _L), jnp.float32),
            pltpu.VMEM_SHARED((_N, FT), jnp.float32),
            pltpu.SemaphoreType.DMA,
            pltpu.SemaphoreType.DMA,
        ],
        compiler_params=pltpu.CompilerParams(use_tc_tiling_on_sc=False),
    )
    def edge_phase(h_hbm, edges_hbm, table_hbm, out_hbm,
                   eidx0_v, eidx1_v, rows0_v, rows1_v, drow0_v, drow1_v,
                   accum_sh, sem0, sem1):
        core = lax.axis_index("c")
        sub = lax.axis_index("s")
        wid = sub * _NC + core
        lane = lax.iota(jnp.int32, _L)
        bufs = ((eidx0_v, rows0_v, drow0_v, sem0),
                (eidx1_v, rows1_v, drow1_v, sem1))

        @pl.loop(0, _K)
        def _(r):
            @pl.loop(0, FT, step=_L)
            def _(cidx):
                rows0_v[r, pl.ds(cidx, _L)] = jnp.zeros((_L,), jnp.float32)

        @pl.loop(0, ROWS_T // ZR)
        def _(i):
            pltpu.sync_copy(rows0_v.at[pl.ds(0, ZR)],
                            accum_sh.at[pl.ds(sub * ROWS_T + i * ZR, ZR)])
        plsc.subcore_barrier()

        shift_idx = jnp.minimum(lane + H, _L - 1)
        pmask = lane < H

        def start(c, s):
            eidx_v, rows_v, drow_v, sem = bufs[s]
            off = c * _K
            pltpu.sync_copy(edges_hbm.at[:, pl.ds(off, _K)], eidx_v)
            pltpu.async_copy(h_hbm.at[eidx_v.at[0]], rows_v, sem)
            pltpu.async_copy(table_hbm.at[eidx_v.at[1]], drow_v, sem)

        def finish(c, s):
            eidx_v, rows_v, drow_v, sem = bufs[s]
            pltpu.make_async_copy(h_hbm.at[eidx_v.at[0]], rows_v, sem).wait()
            pltpu.make_async_copy(table_hbm.at[eidx_v.at[1]], drow_v, sem).wait()

            @pl.loop(0, _K)
            def _(j):
                sv = rows_v[j, pl.ds(F, _L)]
                dv = drow_v[j, pl.ds(0, _L)]
                dsh = _vgather(dv, shift_idx)
                e = sv + dsh
                e = jnp.maximum(e, 0.2 * e)
                pe = jnp.exp(e)
                for h in range(H):
                    phv = _vgather(pe, jnp.full((_L,), h, jnp.int32))
                    for v in range(FH // _L):
                        col = h * FH + v * _L
                        rows_v[j, pl.ds(col, _L)] = (
                            rows_v[j, pl.ds(col, _L)] * phv)
                rows_v[j, pl.ds(F, _L)] = jnp.where(pmask, pe, 0.0)

            pltpu.sync_copy(rows_v, accum_sh.at[eidx_v.at[1]], add=True)

        @pl.when(wid < CHUNKS)
        def _():
            start(wid, 0)

        @pl.loop(0, CPH)
        def _(t):
            c0 = wid + (2 * t) * W
            c1 = c0 + W
            c2 = c1 + W

            @pl.when(c1 < CHUNKS)
            def _():
                start(c1, 1)

            @pl.when(c0 < CHUNKS)
            def _():
                finish(c0, 0)

            @pl.when(c2 < CHUNKS)
            def _():
                start(c2, 0)

            @pl.when(c1 < CHUNKS)
            def _():
                finish(c1, 1)

        plsc.subcore_barrier()
        pltpu.sync_copy(accum_sh.at[pl.ds(sub * ROWS_T, ROWS_T)],
                        out_hbm.at[core, pl.ds(sub * ROWS_T, ROWS_T)])

    return edge_phase


_edge1 = _make_edge_phase(128, 2)
_edge2 = _make_edge_phase(64, 1)


def _tc1(x, W1, a_src1, a_dst1):
    def body(x_ref, w_ref, as_ref, ad_ref, h_ref, sc_ref):
        h = jnp.dot(x_ref[...], w_ref[...],
                    preferred_element_type=jnp.float32)
        cols = []
        for hh in range(2):
            cols.append(jnp.sum(h[:, hh * 64:(hh + 1) * 64]
                                * as_ref[hh][None, :], axis=1))
        for hh in range(2):
            cols.append(jnp.sum(h[:, hh * 64:(hh + 1) * 64]
                                * ad_ref[hh][None, :], axis=1))
        tab = jnp.stack(cols, axis=1)
        sc16 = jnp.concatenate(
            [tab, jnp.zeros((tab.shape[0], 12), jnp.float32)], axis=1)
        h_ref[...] = jnp.concatenate([h, sc16], axis=1)
        sc_ref[...] = sc16

    return pl.pallas_call(
        body,
        grid=(_N // _NB,),
        in_specs=[pl.BlockSpec((_NB, 128), lambda i: (i, 0)),
                  pl.BlockSpec((128, 128), lambda i: (0, 0)),
                  pl.BlockSpec((2, 64), lambda i: (0, 0)),
                  pl.BlockSpec((2, 64), lambda i: (0, 0))],
        out_specs=[pl.BlockSpec((_NB, 144), lambda i: (i, 0)),
                   pl.BlockSpec((_NB, 16), lambda i: (i, 0))],
        out_shape=[jax.ShapeDtypeStruct((_N, 144), jnp.float32),
                   jax.ShapeDtypeStruct((_N, 16), jnp.float32)],
    )(x, W1, a_src1, a_dst1)


def _tc2(acc, b1, W2, a_src2, a_dst2):
    def body(a_ref, b1_ref, w_ref, as_ref, ad_ref, h_ref, sc_ref):
        g = a_ref[0] + a_ref[1]
        parts = []
        for hh in range(2):
            num = g[:, hh * 64:(hh + 1) * 64]
            den = g[:, 128 + hh:129 + hh]
            parts.append(num / (den + 1e-16))
        out1 = jnp.concatenate(parts, axis=1) + b1_ref[...]
        r = jnp.maximum(out1, 0.0)
        h2 = jnp.dot(r, w_ref[...], preferred_element_type=jnp.float32)
        tab = jnp.stack(
            [jnp.sum(h2 * as_ref[0][None, :], axis=1),
             jnp.sum(h2 * ad_ref[0][None, :], axis=1)], axis=1)
        sc16 = jnp.concatenate(
            [tab, jnp.zeros((tab.shape[0], 14), jnp.float32)], axis=1)
        h_ref[...] = jnp.concatenate([h2, sc16], axis=1)
        sc_ref[...] = sc16

    return pl.pallas_call(
        body,
        grid=(_N // _NB,),
        in_specs=[pl.BlockSpec((2, _NB, 144), lambda i: (0, i, 0)),
                  pl.BlockSpec((1, 128), lambda i: (0, 0)),
                  pl.BlockSpec((128, 64), lambda i: (0, 0)),
                  pl.BlockSpec((1, 64), lambda i: (0, 0)),
                  pl.BlockSpec((1, 64), lambda i: (0, 0))],
        out_specs=[pl.BlockSpec((_NB, 80), lambda i: (i, 0)),
                   pl.BlockSpec((_NB, 16), lambda i: (i, 0))],
        out_shape=[jax.ShapeDtypeStruct((_N, 80), jnp.float32),
                   jax.ShapeDtypeStruct((_N, 16), jnp.float32)],
    )(acc, b1, W2, a_src2, a_dst2)


def _tc3(acc, b2):
    def body(a_ref, b2_ref, z_ref):
        g = a_ref[0] + a_ref[1]
        z_ref[...] = g[:, :64] / (g[:, 64:65] + 1e-16) + b2_ref[...]

    return pl.pallas_call(
        body,
        grid=(_N // _NB,),
        in_specs=[pl.BlockSpec((2, _NB, 80), lambda i: (0, i, 0)),
                  pl.BlockSpec((1, 64), lambda i: (0, 0))],
        out_specs=pl.BlockSpec((_NB, 64), lambda i: (i, 0)),
        out_shape=jax.ShapeDtypeStruct((_N, 64), jnp.float32),
    )(acc, b2)


def kernel(x, edge_index, W1, a_src1, a_dst1, b1, W2, a_src2, a_dst2, b2):
    h1ext, tab1 = _tc1(x, W1, a_src1, a_dst1)
    acc1 = _edge1(h1ext, edge_index, tab1)
    h2ext, tab2 = _tc2(acc1, b1.reshape(1, 128), W2, a_src2, a_dst2)
    acc2 = _edge2(h2ext, edge_index, tab2)
    return _tc3(acc2, b2.reshape(1, 64))
